# Initial kernel scaffold; baseline (speedup 1.0000x reference)
import jax
import jax.numpy as jnp
from jax.experimental import pallas as pl


def kernel(x, Wqkv, Wout):
    raise NotImplementedError("write your pallas kernel here")


# speedup vs baseline: 2.5089x; 2.5089x over previous
import functools

import jax
import jax.numpy as jnp
from jax.experimental import pallas as pl
from jax.experimental.pallas import tpu as pltpu

HIDDEN = 1024
HEADS = 16
DH = 64
SEG = 128
SCALE = DH ** -0.5


def _fused_kernel(x_ref, wqkvT_ref, woutT_ref, out_ref, attn_ref):
    x_blk = x_ref[0]
    qkv = jnp.dot(x_blk, wqkvT_ref[...], preferred_element_type=jnp.float32)
    q = qkv[:, :HIDDEN] * SCALE
    k = qkv[:, HIDDEN:2 * HIDDEN]
    v = qkv[:, 2 * HIDDEN:]

    ii = jax.lax.broadcasted_iota(jnp.int32, (64, 64), 0)
    jj = jax.lax.broadcasted_iota(jnp.int32, (64, 64), 1)
    perm = jnp.where(ii + jj == 63, 1.0, 0.0).astype(jnp.float32)
    q_flip = jnp.dot(perm, q[64:, :], preferred_element_type=jnp.float32)
    q2 = jnp.concatenate([q[:64, :], q_flip], axis=0)

    for h in range(HEADS):
        sl = slice(h * DH, (h + 1) * DH)
        qh = q2[:, sl]
        kh = k[:, sl]
        vh = v[:, sl]
        s = jax.lax.dot_general(qh, kh, (((1,), (1,)), ((), ())),
                                preferred_element_type=jnp.float32)
        c0 = jnp.max(s[:64, :], axis=0, keepdims=True)
        c1 = jnp.max(s[64:, :], axis=0, keepdims=True)
        w = jnp.exp(s - jnp.concatenate([jnp.broadcast_to(c0, (64, SEG)),
                                         jnp.broadcast_to(c1, (64, SEG))], axis=0))
        num = jnp.dot(w, vh, preferred_element_type=jnp.float32)
        den = 1e-10 + jnp.sum(w, axis=1, keepdims=True)
        attn_ref[:, sl] = num / den

    out_ref[0] = jnp.dot(attn_ref[...], woutT_ref[...],
                         preferred_element_type=jnp.float32)


@functools.partial(jax.jit, static_argnums=())
def kernel(x, Wqkv, Wout):
    B, M, D = x.shape
    nseg = M // SEG
    wqkvT = Wqkv.T
    woutT = Wout.T

    return pl.pallas_call(
        _fused_kernel,
        grid=(B, nseg),
        in_specs=[
            pl.BlockSpec((1, SEG, HIDDEN), lambda b, s: (b, s, 0)),
            pl.BlockSpec((HIDDEN, 3 * HIDDEN), lambda b, s: (0, 0)),
            pl.BlockSpec((HIDDEN, HIDDEN), lambda b, s: (0, 0)),
        ],
        out_specs=pl.BlockSpec((1, SEG, HIDDEN), lambda b, s: (b, s, 0)),
        out_shape=jax.ShapeDtypeStruct((B, M, D), jnp.float32),
        scratch_shapes=[pltpu.VMEM((SEG, HIDDEN), jnp.float32)],
        compiler_params=pltpu.CompilerParams(
            dimension_semantics=("parallel", "parallel")),
    )(x, wqkvT, woutT)


# speedup vs baseline: 2.6154x; 1.0425x over previous
import functools

import jax
import jax.numpy as jnp
from jax.experimental import pallas as pl
from jax.experimental.pallas import tpu as pltpu

HIDDEN = 1024
HEADS = 16
DH = 64
SEG = 128
SCALE = DH ** -0.5


def _fused_kernel(x_ref, wqkvT_ref, woutT_ref, out_ref, attn_ref):
    x_blk = x_ref[0].astype(jnp.bfloat16)
    qkv = jnp.dot(x_blk, wqkvT_ref[...], preferred_element_type=jnp.float32)
    q = (qkv[:, :HIDDEN] * SCALE).astype(jnp.bfloat16)
    k = qkv[:, HIDDEN:2 * HIDDEN].astype(jnp.bfloat16)
    v = qkv[:, 2 * HIDDEN:].astype(jnp.bfloat16)

    ii = jax.lax.broadcasted_iota(jnp.int32, (64, 64), 0)
    jj = jax.lax.broadcasted_iota(jnp.int32, (64, 64), 1)
    perm = jnp.where(ii + jj == 63, 1.0, 0.0).astype(jnp.bfloat16)
    q_flip = jnp.dot(perm, q[64:, :],
                     preferred_element_type=jnp.float32).astype(jnp.bfloat16)
    q2 = jnp.concatenate([q[:64, :], q_flip], axis=0)

    for h in range(HEADS):
        sl = slice(h * DH, (h + 1) * DH)
        qh = q2[:, sl]
        kh = k[:, sl]
        vh = v[:, sl]
        s = jax.lax.dot_general(qh, kh, (((1,), (1,)), ((), ())),
                                preferred_element_type=jnp.float32)
        c0 = jnp.max(s[:64, :], axis=0, keepdims=True)
        c1 = jnp.max(s[64:, :], axis=0, keepdims=True)
        w = jnp.exp(s - jnp.concatenate([jnp.broadcast_to(c0, (64, SEG)),
                                         jnp.broadcast_to(c1, (64, SEG))], axis=0))
        num = jnp.dot(w.astype(jnp.bfloat16), vh,
                      preferred_element_type=jnp.float32)
        den = 1e-10 + jnp.sum(w, axis=1, keepdims=True)
        attn_ref[:, sl] = (num / den).astype(jnp.bfloat16)

    out_ref[0] = jnp.dot(attn_ref[...], woutT_ref[...],
                         preferred_element_type=jnp.float32)


@functools.partial(jax.jit, static_argnums=())
def kernel(x, Wqkv, Wout):
    B, M, D = x.shape
    nseg = M // SEG
    wqkvT = Wqkv.T.astype(jnp.bfloat16)
    woutT = Wout.T.astype(jnp.bfloat16)

    return pl.pallas_call(
        _fused_kernel,
        grid=(B, nseg),
        in_specs=[
            pl.BlockSpec((1, SEG, HIDDEN), lambda b, s: (b, s, 0)),
            pl.BlockSpec((HIDDEN, 3 * HIDDEN), lambda b, s: (0, 0)),
            pl.BlockSpec((HIDDEN, HIDDEN), lambda b, s: (0, 0)),
        ],
        out_specs=pl.BlockSpec((1, SEG, HIDDEN), lambda b, s: (b, s, 0)),
        out_shape=jax.ShapeDtypeStruct((B, M, D), jnp.float32),
        scratch_shapes=[pltpu.VMEM((SEG, HIDDEN), jnp.bfloat16)],
        compiler_params=pltpu.CompilerParams(
            dimension_semantics=("parallel", "parallel")),
    )(x, wqkvT, woutT)


# speedup vs baseline: 2.8043x; 1.0722x over previous
import functools

import jax
import jax.numpy as jnp
from jax.experimental import pallas as pl
from jax.experimental.pallas import tpu as pltpu

HIDDEN = 1024
HEADS = 16
DH = 64
SEG = 128
SCALE = DH ** -0.5


SEGS_PER_BLK = 2
BLK = SEG * SEGS_PER_BLK


def _fused_kernel(x_ref, wqkvT_ref, woutT_ref, out_ref, attn_ref):
    x_blk = x_ref[0].astype(jnp.bfloat16)
    qkv = jnp.dot(x_blk, wqkvT_ref[...], preferred_element_type=jnp.float32)
    q = (qkv[:, :HIDDEN] * SCALE).astype(jnp.bfloat16)
    k = qkv[:, HIDDEN:2 * HIDDEN].astype(jnp.bfloat16)
    v = qkv[:, 2 * HIDDEN:].astype(jnp.bfloat16)

    ii = jax.lax.broadcasted_iota(jnp.int32, (64, 64), 0)
    jj = jax.lax.broadcasted_iota(jnp.int32, (64, 64), 1)
    perm = jnp.where(ii + jj == 63, 1.0, 0.0).astype(jnp.bfloat16)

    for g in range(SEGS_PER_BLK):
        r0 = g * SEG
        q_flip = jnp.dot(perm, q[r0 + 64:r0 + SEG, :],
                         preferred_element_type=jnp.float32).astype(jnp.bfloat16)
        q2 = jnp.concatenate([q[r0:r0 + 64, :], q_flip], axis=0)
        for h in range(HEADS):
            sl = slice(h * DH, (h + 1) * DH)
            qh = q2[:, sl]
            kh = k[r0:r0 + SEG, sl]
            vh = v[r0:r0 + SEG, sl]
            s = jax.lax.dot_general(qh, kh, (((1,), (1,)), ((), ())),
                                    preferred_element_type=jnp.float32)
            c0 = jnp.max(s[:64, :], axis=0, keepdims=True)
            c1 = jnp.max(s[64:, :], axis=0, keepdims=True)
            w = jnp.exp(s - jnp.concatenate(
                [jnp.broadcast_to(c0, (64, SEG)),
                 jnp.broadcast_to(c1, (64, SEG))], axis=0))
            num = jnp.dot(w.astype(jnp.bfloat16), vh,
                          preferred_element_type=jnp.float32)
            den = 1e-10 + jnp.sum(w, axis=1, keepdims=True)
            attn_ref[r0:r0 + SEG, sl] = (num / den).astype(jnp.bfloat16)

    out_ref[0] = jnp.dot(attn_ref[...], woutT_ref[...],
                         preferred_element_type=jnp.float32)


@functools.partial(jax.jit, static_argnums=())
def kernel(x, Wqkv, Wout):
    B, M, D = x.shape
    nseg = M // SEG
    wqkvT = Wqkv.T.astype(jnp.bfloat16)
    woutT = Wout.T.astype(jnp.bfloat16)

    return pl.pallas_call(
        _fused_kernel,
        grid=(B, nseg // SEGS_PER_BLK),
        in_specs=[
            pl.BlockSpec((1, BLK, HIDDEN), lambda b, s: (b, s, 0)),
            pl.BlockSpec((HIDDEN, 3 * HIDDEN), lambda b, s: (0, 0)),
            pl.BlockSpec((HIDDEN, HIDDEN), lambda b, s: (0, 0)),
        ],
        out_specs=pl.BlockSpec((1, BLK, HIDDEN), lambda b, s: (b, s, 0)),
        out_shape=jax.ShapeDtypeStruct((B, M, D), jnp.float32),
        scratch_shapes=[pltpu.VMEM((BLK, HIDDEN), jnp.bfloat16)],
        compiler_params=pltpu.CompilerParams(
            dimension_semantics=("parallel", "parallel")),
    )(x, wqkvT, woutT)
